# Initial kernel scaffold; baseline (speedup 1.0000x reference)
import functools

import jax
import jax.numpy as jnp
from jax import lax
from jax.experimental import pallas as pl
from jax.experimental.pallas import tpu as pltpu
from jax.experimental.pallas import tpu_sc as plsc

N = 10000
E = 320000
D = 128
H = 32
_SIMILAR_WEIGHT = 1.0

NC = 2
NS = 16
L = 16
NW = NC * NS
EPW = E // NW
CH = 80
NCH = EPW // CH
RPT = N // NS

ROW_BLK = 1000


def _dense_body(x_ref, wp_ref, ws_ref, b_ref, y_ref, z_ref):
    x = x_ref[...]
    y_ref[...] = jnp.dot(x, wp_ref[...], preferred_element_type=jnp.float32)
    z_ref[...] = (
        jnp.dot(x, ws_ref[...], preferred_element_type=jnp.float32) + b_ref[...]
    )


def _dense(X, W_pass, W_self, bias):
    return pl.pallas_call(
        _dense_body,
        grid=(N // ROW_BLK,),
        in_specs=[
            pl.BlockSpec((ROW_BLK, D), lambda i: (i, 0)),
            pl.BlockSpec((D, H), lambda i: (0, 0)),
            pl.BlockSpec((D, H), lambda i: (0, 0)),
            pl.BlockSpec((1, H), lambda i: (0, 0)),
        ],
        out_specs=[
            pl.BlockSpec((ROW_BLK, H), lambda i: (i, 0)),
            pl.BlockSpec((ROW_BLK, H), lambda i: (i, 0)),
        ],
        out_shape=[
            jax.ShapeDtypeStruct((N, H), jnp.float32),
            jax.ShapeDtypeStruct((N, H), jnp.float32),
        ],
    )(X, W_pass, W_self, bias)


_sc_mesh = plsc.VectorSubcoreMesh(core_axis_name="c", subcore_axis_name="s")


@functools.partial(
    pl.kernel,
    out_type=jax.ShapeDtypeStruct((NC, N, H), jnp.float32),
    mesh=_sc_mesh,
    scratch_types=[
        pltpu.VMEM((NCH, CH), jnp.int32),
        pltpu.VMEM((NCH, CH), jnp.int32),
        pltpu.VMEM((NCH, CH), jnp.float32),
        pltpu.VMEM((CH, H), jnp.float32),
        pltpu.VMEM((RPT, H), jnp.float32),
        pltpu.VMEM_SHARED((N, H), jnp.float32),
    ],
)
def _sc_segsum(y_hbm, dsti_hbm, srci_hbm, ev_hbm, out_hbm,
               dsti_v, srci_v, ev_v, rows_v, zero_v, acc_sh):
    cid = lax.axis_index("c")
    sid = lax.axis_index("s")
    wid = sid * NC + cid

    def _zrow(i, carry):
        zero_v[i, 0:L] = jnp.zeros((L,), jnp.float32)
        zero_v[i, L:2 * L] = jnp.zeros((L,), jnp.float32)
        return carry

    lax.fori_loop(0, RPT, _zrow, 0)
    pltpu.sync_copy(zero_v, acc_sh.at[pl.ds(sid * RPT, RPT)])
    plsc.subcore_barrier()

    rbase = wid * NCH
    pltpu.sync_copy(dsti_hbm.at[pl.ds(rbase, NCH)], dsti_v)
    pltpu.sync_copy(srci_hbm.at[pl.ds(rbase, NCH)], srci_v)
    pltpu.sync_copy(ev_hbm.at[pl.ds(rbase, NCH)], ev_v)

    iota16 = lax.iota(jnp.int32, L)

    def _chunk(j, carry):
        pltpu.sync_copy(y_hbm.at[dsti_v.at[j]], rows_v)

        def _grp(g, c2):
            e16 = iota16 + g * L
            ev16 = ev_v[j, pl.ds(g * L, L)]
            for d in range(H):
                col = jnp.full((L,), d, jnp.int32)
                v = plsc.load_gather(rows_v, [e16, col])
                plsc.store_scatter(rows_v, [e16, col], v * ev16)
            return c2

        lax.fori_loop(0, CH // L, _grp, 0)
        pltpu.sync_copy(rows_v, acc_sh.at[srci_v.at[j]], add=True)
        return carry

    lax.fori_loop(0, NCH, _chunk, 0)

    plsc.subcore_barrier()
    pltpu.sync_copy(
        acc_sh.at[pl.ds(sid * RPT, RPT)],
        out_hbm.at[cid, pl.ds(sid * RPT, RPT)],
    )


def _combine_body(p_ref, z_ref, hx_ref):
    hx_ref[...] = jnp.maximum(p_ref[0] + p_ref[1] + z_ref[...], 0.0)


def _combine(partials, Z):
    return pl.pallas_call(
        _combine_body,
        grid=(N // ROW_BLK,),
        in_specs=[
            pl.BlockSpec((NC, ROW_BLK, H), lambda i: (0, i, 0)),
            pl.BlockSpec((ROW_BLK, H), lambda i: (i, 0)),
        ],
        out_specs=pl.BlockSpec((ROW_BLK, H), lambda i: (i, 0)),
        out_shape=jax.ShapeDtypeStruct((N, H), jnp.float32),
    )(partials, Z)


@functools.partial(
    pl.kernel,
    out_type=jax.ShapeDtypeStruct((NW * NCH, CH), jnp.float32),
    mesh=_sc_mesh,
    scratch_types=[
        pltpu.VMEM((NCH, CH), jnp.int32),
        pltpu.VMEM((NCH, CH), jnp.int32),
        pltpu.VMEM((CH, H), jnp.float32),
        pltpu.VMEM((CH, H), jnp.float32),
        pltpu.VMEM((NCH, CH), jnp.float32),
    ],
)
def _sc_edgedot(hx_hbm, srci_hbm, dsti_hbm, out_hbm,
                srci_v, dsti_v, srows_v, drows_v, dots_v):
    cid = lax.axis_index("c")
    sid = lax.axis_index("s")
    wid = sid * NC + cid

    rbase = wid * NCH
    pltpu.sync_copy(srci_hbm.at[pl.ds(rbase, NCH)], srci_v)
    pltpu.sync_copy(dsti_hbm.at[pl.ds(rbase, NCH)], dsti_v)

    iota16 = lax.iota(jnp.int32, L)

    def _chunk(j, carry):
        pltpu.sync_copy(hx_hbm.at[srci_v.at[j]], srows_v)
        pltpu.sync_copy(hx_hbm.at[dsti_v.at[j]], drows_v)

        def _grp(g, c2):
            e16 = iota16 + g * L
            acc = jnp.zeros((L,), jnp.float32)
            for d in range(H):
                col = jnp.full((L,), d, jnp.int32)
                a = plsc.load_gather(srows_v, [e16, col])
                b = plsc.load_gather(drows_v, [e16, col])
                acc = acc + a * b
            dots_v[j, pl.ds(g * L, L)] = acc
            return c2

        lax.fori_loop(0, CH // L, _grp, 0)
        return carry

    lax.fori_loop(0, NCH, _chunk, 0)
    pltpu.sync_copy(dots_v, out_hbm.at[pl.ds(rbase, NCH)])


def _loss_body(s_ref, c_ref, o_ref):
    s = s_ref[...]
    sx = 1.0 / (1.0 + jnp.exp(-s))
    sxc = jnp.clip(sx, 1e-12, 1.0 - 1e-7)
    w = jnp.where(sxc < 0.5, _SIMILAR_WEIGHT, 1.0)
    cf = c_ref[...].astype(jnp.float32)
    v = w * -(cf * jnp.log(sxc) + (1.0 - cf) * jnp.log(1.0 - sxc))
    o_ref[...] = (jnp.sum(v) / E).reshape(1, 1)


def _loss(logits2d, c2d):
    return pl.pallas_call(
        _loss_body,
        out_shape=jax.ShapeDtypeStruct((1, 1), jnp.float32),
    )(logits2d, c2d)


def kernel(X, edge_values, W_pass, b_pass, W_self, b_self, edge_index, C):
    src = edge_index[0]
    dst = edge_index[1]
    bias = (b_pass + b_self).reshape(1, H)

    Y, Z = _dense(X, W_pass, W_self, bias)

    src2 = src.reshape(NW * NCH, CH)
    dst2 = dst.reshape(NW * NCH, CH)
    ev2 = edge_values.reshape(NW * NCH, CH)

    partials = _sc_segsum(Y, dst2, src2, ev2)
    Hx = _combine(partials, Z)
    logits = _sc_edgedot(Hx, src2, dst2)

    loss = _loss(logits.reshape(E // D, D), C.reshape(E // D, D))
    return loss[0, 0]


# speedup vs baseline: 7.0940x; 7.0940x over previous
import functools

import jax
import jax.numpy as jnp
from jax import lax
from jax.experimental import pallas as pl
from jax.experimental.pallas import tpu as pltpu
from jax.experimental.pallas import tpu_sc as plsc

N = 10000
E = 320000
D = 128
H = 32
_SIMILAR_WEIGHT = 1.0

NC = 2
NS = 16
L = 16
NW = NC * NS
EPW = E // NW
CH = 80
NCH = EPW // CH
NP = 10240
RPT = NP // NS

ROW_BLK = 1000


def _dyn_gather16(vec, idx16):
    return lax.gather(
        vec,
        idx16[:, None],
        dimension_numbers=lax.GatherDimensionNumbers(
            offset_dims=(), collapsed_slice_dims=(0,), start_index_map=(0,)
        ),
        slice_sizes=(1,),
        mode=lax.GatherScatterMode.PROMISE_IN_BOUNDS,
    )


def _dense_body(x_ref, wp_ref, ws_ref, b_ref, y_ref, z_ref):
    x = x_ref[...]
    y_ref[...] = jnp.dot(x, wp_ref[...], preferred_element_type=jnp.float32)
    z_ref[...] = (
        jnp.dot(x, ws_ref[...], preferred_element_type=jnp.float32) + b_ref[...]
    )


def _dense(X, W_pass, W_self, bias):
    return pl.pallas_call(
        _dense_body,
        grid=(N // ROW_BLK,),
        in_specs=[
            pl.BlockSpec((ROW_BLK, D), lambda i: (i, 0)),
            pl.BlockSpec((D, H), lambda i: (0, 0)),
            pl.BlockSpec((D, H), lambda i: (0, 0)),
            pl.BlockSpec((1, H), lambda i: (0, 0)),
        ],
        out_specs=[
            pl.BlockSpec((ROW_BLK, H), lambda i: (i, 0)),
            pl.BlockSpec((ROW_BLK, H), lambda i: (i, 0)),
        ],
        out_shape=[
            jax.ShapeDtypeStruct((N, H), jnp.float32),
            jax.ShapeDtypeStruct((N, H), jnp.float32),
        ],
    )(X, W_pass, W_self, bias)


@functools.cache
def _make_sc_segsum():
    mesh = plsc.VectorSubcoreMesh(core_axis_name="c", subcore_axis_name="s")
    return functools.partial(
        pl.kernel,
        out_type=jax.ShapeDtypeStruct((NC, NP, H), jnp.float32),
        mesh=mesh,
        compiler_params=pltpu.CompilerParams(use_tc_tiling_on_sc=False),
        scratch_types=[
            pltpu.VMEM((NCH, CH), jnp.int32),
            pltpu.VMEM((NCH, CH), jnp.int32),
            pltpu.VMEM((NCH, CH), jnp.float32),
            pltpu.VMEM((CH, H), jnp.float32),
            pltpu.VMEM((RPT, H), jnp.float32),
            pltpu.VMEM_SHARED((NP, H), jnp.float32),
        ],
    )(_sc_segsum_body)


def _sc_segsum_body(y_hbm, dsti_hbm, srci_hbm, ev_hbm, out_hbm,
                    dsti_v, srci_v, ev_v, rows_v, zero_v, acc_sh):
    cid = lax.axis_index("c")
    sid = lax.axis_index("s")
    wid = sid * NC + cid

    def _zrow(i, carry):
        zero_v[i, 0:L] = jnp.zeros((L,), jnp.float32)
        zero_v[i, L:2 * L] = jnp.zeros((L,), jnp.float32)
        return carry

    lax.fori_loop(0, RPT, _zrow, 0)
    pltpu.sync_copy(zero_v, acc_sh.at[pl.ds(sid * RPT, RPT)])
    plsc.subcore_barrier()

    pltpu.sync_copy(dsti_hbm.at[wid], dsti_v)
    pltpu.sync_copy(srci_hbm.at[wid], srci_v)
    pltpu.sync_copy(ev_hbm.at[wid], ev_v)

    iota16 = lax.iota(jnp.int32, L)

    def _chunk(j, carry):
        pltpu.sync_copy(y_hbm.at[dsti_v.at[j]], rows_v)

        def _grp(g, c2):
            ev16 = ev_v[j, pl.ds(g * L, L)]

            def _edge(k, c3):
                i = g * L + k
                evb = _dyn_gather16(ev16, jnp.broadcast_to(k, (L,)))
                rows_v[i, 0:L] = rows_v[i, 0:L] * evb
                rows_v[i, L:2 * L] = rows_v[i, L:2 * L] * evb
                return c3

            return lax.fori_loop(0, L, _edge, c2)

        lax.fori_loop(0, CH // L, _grp, 0)
        pltpu.sync_copy(rows_v, acc_sh.at[srci_v.at[j]], add=True)
        return carry

    lax.fori_loop(0, NCH, _chunk, 0)

    plsc.subcore_barrier()
    pltpu.sync_copy(
        acc_sh.at[pl.ds(sid * RPT, RPT)],
        out_hbm.at[cid, pl.ds(sid * RPT, RPT)],
    )


def _combine_body(p_ref, z_ref, hx_ref):
    hx_ref[...] = jnp.maximum(p_ref[0] + p_ref[1] + z_ref[...], 0.0)


def _combine(partials, Z):
    return pl.pallas_call(
        _combine_body,
        grid=(N // ROW_BLK,),
        in_specs=[
            pl.BlockSpec((NC, ROW_BLK, H), lambda i: (0, i, 0)),
            pl.BlockSpec((ROW_BLK, H), lambda i: (i, 0)),
        ],
        out_specs=pl.BlockSpec((ROW_BLK, H), lambda i: (i, 0)),
        out_shape=jax.ShapeDtypeStruct((N, H), jnp.float32),
    )(partials, Z)


@functools.cache
def _make_sc_edgedot():
    mesh = plsc.VectorSubcoreMesh(core_axis_name="c", subcore_axis_name="s")
    return functools.partial(
        pl.kernel,
        out_type=jax.ShapeDtypeStruct((NW, NCH, CH), jnp.float32),
        mesh=mesh,
        compiler_params=pltpu.CompilerParams(use_tc_tiling_on_sc=False),
        scratch_types=[
            pltpu.VMEM((NCH, CH), jnp.int32),
            pltpu.VMEM((NCH, CH), jnp.int32),
            pltpu.VMEM((CH, H), jnp.float32),
            pltpu.VMEM((CH, H), jnp.float32),
            pltpu.VMEM((NCH, CH), jnp.float32),
        ],
    )(_sc_edgedot_body)


def _sc_edgedot_body(hx_hbm, srci_hbm, dsti_hbm, out_hbm,
                     srci_v, dsti_v, srows_v, drows_v, dots_v):
    cid = lax.axis_index("c")
    sid = lax.axis_index("s")
    wid = sid * NC + cid

    pltpu.sync_copy(srci_hbm.at[wid], srci_v)
    pltpu.sync_copy(dsti_hbm.at[wid], dsti_v)

    iota16 = lax.iota(jnp.int32, L)

    def _chunk(j, carry):
        pltpu.sync_copy(hx_hbm.at[srci_v.at[j]], srows_v)
        pltpu.sync_copy(hx_hbm.at[dsti_v.at[j]], drows_v)

        def _grp(g, c2):
            def _edge(k, acc):
                i = g * L + k
                v = (srows_v[i, 0:L] * drows_v[i, 0:L]
                     + srows_v[i, L:2 * L] * drows_v[i, L:2 * L])
                for sh in (1, 2, 4, 8):
                    v = v + _dyn_gather16(v, jnp.bitwise_xor(iota16, sh))
                return jnp.where(iota16 == k, v, acc)

            acc = lax.fori_loop(0, L, _edge, jnp.zeros((L,), jnp.float32))
            dots_v[j, pl.ds(g * L, L)] = acc
            return c2

        lax.fori_loop(0, CH // L, _grp, 0)
        return carry

    lax.fori_loop(0, NCH, _chunk, 0)
    pltpu.sync_copy(dots_v, out_hbm.at[wid])


def _loss_body(s_ref, c_ref, o_ref):
    s = s_ref[...]
    sx = 1.0 / (1.0 + jnp.exp(-s))
    sxc = jnp.clip(sx, 1e-12, 1.0 - 1e-7)
    w = jnp.where(sxc < 0.5, _SIMILAR_WEIGHT, 1.0)
    cf = c_ref[...].astype(jnp.float32)
    v = w * -(cf * jnp.log(sxc) + (1.0 - cf) * jnp.log(1.0 - sxc))
    o_ref[...] = (jnp.sum(v) / E).reshape(1, 1)


def _loss(logits2d, c2d):
    return pl.pallas_call(
        _loss_body,
        out_shape=jax.ShapeDtypeStruct((1, 1), jnp.float32),
    )(logits2d, c2d)


def kernel(X, edge_values, W_pass, b_pass, W_self, b_self, edge_index, C):
    src = edge_index[0]
    dst = edge_index[1]
    bias = (b_pass + b_self).reshape(1, H)

    Y, Z = _dense(X, W_pass, W_self, bias)

    src2 = src.reshape(NW, NCH, CH)
    dst2 = dst.reshape(NW, NCH, CH)
    ev2 = edge_values.reshape(NW, NCH, CH)

    partials = _make_sc_segsum()(Y, dst2, src2, ev2)
    Hx = _combine(partials[:, :N], Z)
    logits = _make_sc_edgedot()(Hx, src2, dst2)

    loss = _loss(logits.reshape(E // D, D), C.reshape(E // D, D))
    return loss[0, 0]
